# baseline (speedup 1.0000x reference)
import jax
import jax.numpy as jnp
from jax import lax
from jax.experimental import pallas as pl
from jax.experimental.pallas import tpu as pltpu
from jax.experimental.pallas import tpu_sc as plsc

L = 16
NC = 2
NS = 16
NW = NC * NS
NCOL = 64
NROW_X = 1000000
ALIGNED_ROWS = 999936
TAIL = NROW_X - ALIGNED_ROWS
NROWS_OUT = 16384
TOTAL = NROWS_OUT * NCOL
E = TOTAL // NW
GROUP = 128
NG = E // GROUP
TPB = 8192
QT = 2048
Q3T = 7812 - 3 * QT
IMG_ROWS = NCOL * TPB
IMG = IMG_ROWS * 128


def _copy_body(xt_hbm, tailp_hbm, img_hbm, sem):
    cid = lax.axis_index("c")
    sid = lax.axis_index("s")
    band = cid * 4 + sid // 4
    q = sid % 4
    j0 = pl.multiple_of(band * 8, 8)
    t0 = q * QT
    nt = jnp.where(q == 3, Q3T, QT)
    row0 = band * TPB + t0

    def fire(t, carry):
        pltpu.async_copy(
            xt_hbm.at[pl.ds(j0, 8), pl.ds((t0 + t) * 128, 128)],
            img_hbm.at[pl.ds((row0 + t) * 8, 8), :],
            sem,
        )
        return carry

    lax.fori_loop(0, nt, fire, 0)

    @pl.when(q == 3)
    def _():
        pltpu.async_copy(
            tailp_hbm.at[pl.ds(j0, 8), :],
            img_hbm.at[pl.ds((band * TPB + 7812) * 8, 8), :],
            sem,
        )

    def drain(t, carry):
        pltpu.make_async_copy(
            img_hbm.at[pl.ds(0, 8), :], img_hbm.at[pl.ds(8, 8), :], sem
        ).wait()
        return carry

    lax.fori_loop(0, nt + jnp.where(q == 3, 1, 0), drain, 0)


def _gather_body(img_hbm, idx_hbm, out_hbm, fidx_v, out_v, sem):
    cid = lax.axis_index("c")
    sid = lax.axis_index("s")
    g0 = cid * (NCOL // NC) + 2 * sid
    base = g0 * NROWS_OUT
    pltpu.sync_copy(idx_hbm.at[pl.ds(base, E)], fidx_v)

    def compute(col, carry):
        j = g0 + col
        cj = jnp.full((L,), ((j >> 3) << 23) + ((j & 7) << 7), jnp.int32)
        run0 = col * NROWS_OUT

        def add_chunk(g, carry):
            p = run0 + g * L
            iv = fidx_v[pl.ds(p, L)]
            fidx_v[pl.ds(p, L)] = ((iv >> 7) << 10) + (iv & 127) + cj
            return carry

        return lax.fori_loop(0, NROWS_OUT // L, add_chunk, carry)

    lax.fori_loop(0, 2, compute, 0)

    def fire(r, carry):
        pltpu.async_copy(
            img_hbm.at[fidx_v.at[pl.ds(r * GROUP, GROUP)]],
            out_v.at[pl.ds(r * GROUP, GROUP)],
            sem,
        )
        return carry

    lax.fori_loop(0, NG, fire, 0)
    pltpu.make_async_copy(img_hbm.at[pl.ds(0, E)], out_v, sem).wait()

    pltpu.sync_copy(out_v, out_hbm.at[pl.ds(base, E)])


def kernel(x, dim, index):
    del dim
    mesh = plsc.VectorSubcoreMesh(core_axis_name="c", subcore_axis_name="s")
    params = pltpu.CompilerParams(needs_layout_passes=False)

    xt = x.T
    idxf = index.astype(jnp.int32).T.reshape(-1)
    tailp = jnp.pad(x[ALIGNED_ROWS:].T, ((0, 0), (0, 128 - TAIL)))

    img2d = pl.kernel(
        _copy_body,
        out_type=jax.ShapeDtypeStruct((IMG_ROWS, 128), jnp.float32),
        mesh=mesh,
        compiler_params=params,
        scratch_types=[pltpu.SemaphoreType.DMA],
    )(xt, tailp)
    imgf = img2d.reshape(-1)

    outf = pl.kernel(
        _gather_body,
        out_type=jax.ShapeDtypeStruct((TOTAL,), jnp.float32),
        mesh=mesh,
        compiler_params=params,
        scratch_types=[
            pltpu.VMEM((E,), jnp.int32),
            pltpu.VMEM((E,), jnp.float32),
            pltpu.SemaphoreType.DMA,
        ],
    )(imgf, idxf)
    return outf.reshape(NCOL, NROWS_OUT).T


# speedup vs baseline: 11.5903x; 11.5903x over previous
import jax
import jax.numpy as jnp
from jax import lax
from jax.experimental import pallas as pl
from jax.experimental.pallas import tpu as pltpu
from jax.experimental.pallas import tpu_sc as plsc

L = 16
NC = 2
NS = 16
NW = NC * NS
NCOL = 64
NROW_X = 1000000
NROWS_OUT = 16384
TOTAL = NROWS_OUT * NCOL
E = TOTAL // NW
GROUP = 128
NG = E // GROUP


def _gather_body(x_hbm, idx_hbm, out_hbm, fidx_v, out_v, sem):
    cid = lax.axis_index("c")
    sid = lax.axis_index("s")
    g0 = cid * (NCOL // NC) + 2 * sid
    base = g0 * NROWS_OUT
    pltpu.sync_copy(idx_hbm.at[pl.ds(base, E)], fidx_v)

    def compute(col, carry):
        off = jnp.full((L,), g0 + col, jnp.int32)
        run0 = col * NROWS_OUT

        def add_chunk(g, carry):
            p = run0 + g * L
            fidx_v[pl.ds(p, L)] = fidx_v[pl.ds(p, L)] * NCOL + off
            return carry

        return lax.fori_loop(0, NROWS_OUT // L, add_chunk, carry)

    lax.fori_loop(0, 2, compute, 0)

    def fire(r, carry):
        pltpu.async_copy(
            x_hbm.at[fidx_v.at[pl.ds(r * GROUP, GROUP)]],
            out_v.at[pl.ds(r * GROUP, GROUP)],
            sem,
        )
        return carry

    lax.fori_loop(0, NG, fire, 0)
    pltpu.make_async_copy(x_hbm.at[pl.ds(0, E)], out_v, sem).wait()

    pltpu.sync_copy(out_v, out_hbm.at[pl.ds(base, E)])


def kernel(x, dim, index):
    del dim
    x2 = lax.optimization_barrier(x.reshape(NROW_X // 2, 2 * NCOL))
    xf = x2.reshape(-1)
    idxf = index.astype(jnp.int32).T.reshape(-1)
    outf = pl.kernel(
        _gather_body,
        out_type=jax.ShapeDtypeStruct((TOTAL,), jnp.float32),
        mesh=plsc.VectorSubcoreMesh(core_axis_name="c", subcore_axis_name="s"),
        compiler_params=pltpu.CompilerParams(needs_layout_passes=False),
        scratch_types=[
            pltpu.VMEM((E,), jnp.int32),
            pltpu.VMEM((E,), jnp.float32),
            pltpu.SemaphoreType.DMA,
        ],
    )(xf, idxf)
    return outf.reshape(NCOL, NROWS_OUT).T
